# baseline (speedup 1.0000x reference)
import functools

import jax
import jax.numpy as jnp
from jax import lax
from jax.experimental import pallas as pl
from jax.experimental.pallas import tpu as pltpu
from jax.experimental.pallas import tpu_sc as plsc

N = 10000
H = 128
NC = 2
NS = 16
NW = NC * NS
K = 128
NCH = 80
NCHH = NCH // 2
E_TILE = NCH * K
E_PAD = NW * E_TILE
PN = 10112
RPT = PN // NS

_mesh = plsc.VectorSubcoreMesh(core_axis_name="c", subcore_axis_name="s")


NB = 2


@functools.partial(
    pl.kernel,
    mesh=_mesh,
    out_type=jax.ShapeDtypeStruct((NC, PN, H), jnp.float32),
    scratch_types=[
        pltpu.VMEM_SHARED((PN, H), jnp.float32),
        pltpu.VMEM((NCHH, K), jnp.int32),
        pltpu.VMEM((NCHH, K), jnp.int32),
        pltpu.VMEM((K, H), jnp.float32),
        pltpu.VMEM((K, H), jnp.float32),
        pltpu.SemaphoreType.DMA,
        pltpu.SemaphoreType.DMA,
        pltpu.SemaphoreType.DMA,
        pltpu.SemaphoreType.DMA,
    ],
)
def _sc_spmm(g_hbm, e_hbm, z_hbm, out_hbm, acc_sh, src_v, dst_v, b0, b1,
             g0, g1, s0, s1):
    bufs = [b0, b1]
    gsem = [g0, g1]
    ssem = [s0, s1]
    c = lax.axis_index("c")
    s = lax.axis_index("s")
    w = c * NS + s
    pltpu.sync_copy(z_hbm.at[pl.ds(s * RPT, RPT)], acc_sh.at[pl.ds(s * RPT, RPT)])
    plsc.subcore_barrier()

    for h in range(2):
        pltpu.sync_copy(e_hbm.at[0, w, pl.ds(h * NCHH, NCHH)], src_v)
        pltpu.sync_copy(e_hbm.at[1, w, pl.ds(h * NCHH, NCHH)], dst_v)
        for b in range(NB):
            pltpu.async_copy(g_hbm.at[src_v.at[b]], bufs[b], gsem[b])

        def outer(jo, carry):
            for b in range(NB):
                jj = jo * NB + b
                pltpu.make_async_copy(g_hbm.at[src_v.at[jj]], bufs[b],
                                      gsem[b]).wait()
                pltpu.async_copy(bufs[b], acc_sh.at[dst_v.at[jj]], ssem[b],
                                 add=True)
            for b in range(NB):
                jj = jo * NB + b
                pltpu.make_async_copy(bufs[b], acc_sh.at[dst_v.at[jj]],
                                      ssem[b]).wait()
                pltpu.async_copy(g_hbm.at[src_v.at[jj + NB]], bufs[b], gsem[b])
            return carry

        lax.fori_loop(0, NCHH // NB - 1, outer, 0)
        for b in range(NB):
            jj = NCHH - NB + b
            pltpu.make_async_copy(g_hbm.at[src_v.at[jj]], bufs[b],
                                  gsem[b]).wait()
            pltpu.async_copy(bufs[b], acc_sh.at[dst_v.at[jj]], ssem[b],
                             add=True)
        for b in range(NB):
            jj = NCHH - NB + b
            pltpu.make_async_copy(bufs[b], acc_sh.at[dst_v.at[jj]],
                                  ssem[b]).wait()
    plsc.subcore_barrier()
    pltpu.sync_copy(acc_sh.at[pl.ds(s * RPT, RPT)],
                    out_hbm.at[c, pl.ds(s * RPT, RPT)])


def _prep_body(x_ref, we_ref, be_ref, wc0_ref, degp_ref, g0_ref, dis_ref):
    deg = degp_ref[0, :, 0:1] + degp_ref[1, :, 0:1] + 1.0
    row = lax.broadcasted_iota(jnp.int32, (PN, 1), 0)
    dis = jnp.where(row < N, lax.rsqrt(deg), 0.0)
    h0 = jnp.maximum(
        jnp.dot(x_ref[...], we_ref[...], preferred_element_type=jnp.float32)
        + be_ref[...], 0.0)
    g0_ref[...] = dis * jnp.dot(h0, wc0_ref[...],
                                preferred_element_type=jnp.float32)
    dis_ref[...] = dis


def _mid_body(accp_ref, g_ref, dis_ref, b_ref, wn_ref, gn_ref):
    dis = dis_ref[...]
    agg = accp_ref[0] + accp_ref[1] + g_ref[...]
    h = jnp.maximum(dis * agg + b_ref[...], 0.0)
    gn_ref[...] = dis * jnp.dot(h, wn_ref[...],
                                preferred_element_type=jnp.float32)


def _final_body(accp_ref, g_ref, dis_ref, b_ref, wd_ref, bd_ref, out_ref):
    dis = dis_ref[...]
    agg = accp_ref[0] + accp_ref[1] + g_ref[...]
    h = jnp.maximum(dis * agg + b_ref[...], 0.0)
    out_ref[...] = jnp.dot(h, wd_ref[...],
                           preferred_element_type=jnp.float32) + bd_ref[...]


def _tc(body, out_shapes, *args):
    return pl.pallas_call(body, out_shape=out_shapes)(*args)


def kernel(x, edge_index, W_enc, b_enc, W_conv, b_conv, W_dec, b_dec):
    f32 = jnp.float32
    L = W_conv.shape[0]
    d_out = W_dec.shape[1]

    e_t = edge_index.shape[1] // NW
    real = edge_index.reshape(2, NW, e_t)
    padrow = (N + jnp.arange(E_TILE - e_t, dtype=jnp.int32) % (PN - N))
    pads = jnp.broadcast_to(padrow, (2, NW, E_TILE - e_t))
    e3 = jnp.concatenate([real, pads], axis=2).reshape(2, NW, NCH, K)
    xp = jnp.pad(x, ((0, PN - N), (0, 0)))
    zeros = jnp.zeros((PN, H), f32)
    row = jnp.arange(PN, dtype=jnp.int32)[:, None]
    ones_g = jnp.where(row < N, 1.0, 0.0) * jnp.ones((PN, H), f32)
    be2 = b_enc.reshape(1, H)
    wd_pad = jnp.pad(W_dec, ((0, 0), (0, H - d_out)))
    bd_pad = jnp.pad(b_dec, (0, H - d_out)).reshape(1, H)

    degp = _sc_spmm(ones_g, e3, zeros)
    g, dis = _tc(
        _prep_body,
        (jax.ShapeDtypeStruct((PN, H), f32), jax.ShapeDtypeStruct((PN, 1), f32)),
        xp, W_enc, be2, W_conv[0], degp)

    for i in range(L):
        accp = _sc_spmm(g, e3, zeros)
        bi = b_conv[i].reshape(1, H)
        if i + 1 < L:
            g = _tc(_mid_body, jax.ShapeDtypeStruct((PN, H), f32),
                    accp, g, dis, bi, W_conv[i + 1])
        else:
            outp = _tc(_final_body, jax.ShapeDtypeStruct((PN, H), f32),
                       accp, g, dis, bi, wd_pad, bd_pad)
    return outp[:N, :d_out]


# speedup vs baseline: 1.2663x; 1.2663x over previous
import functools

import jax
import jax.numpy as jnp
from jax import lax
from jax.experimental import pallas as pl
from jax.experimental.pallas import tpu as pltpu
from jax.experimental.pallas import tpu_sc as plsc

N = 10000
H = 128
NC = 2
NS = 16
NW = NC * NS
K = 128
NCH = 80
NCHH = NCH // 2
E_TILE = NCH * K
E_PAD = NW * E_TILE
PN = 10112
RPT = PN // NS

_mesh = plsc.VectorSubcoreMesh(core_axis_name="c", subcore_axis_name="s")


NB = 2


@functools.partial(
    pl.kernel,
    mesh=_mesh,
    out_type=jax.ShapeDtypeStruct((NC, PN, H), jnp.float32),
    scratch_types=[
        pltpu.VMEM_SHARED((PN, H), jnp.float32),
        pltpu.VMEM((NCHH, K), jnp.int32),
        pltpu.VMEM((NCHH, K), jnp.int32),
        pltpu.VMEM((K, H), jnp.float32),
        pltpu.VMEM((K, H), jnp.float32),
        pltpu.SemaphoreType.DMA,
        pltpu.SemaphoreType.DMA,
    ],
)
def _sc_spmm(g_hbm, e_hbm, z_hbm, out_hbm, acc_sh, src_v, dst_v, b0, b1,
             g0, g1):
    bufs = [b0, b1]
    gsem = [g0, g1]
    c = lax.axis_index("c")
    s = lax.axis_index("s")
    w = c * NS + s
    pltpu.sync_copy(z_hbm.at[pl.ds(s * RPT, RPT)], acc_sh.at[pl.ds(s * RPT, RPT)])
    plsc.subcore_barrier()

    for h in range(2):
        pltpu.sync_copy(e_hbm.at[0, w, pl.ds(h * NCHH, NCHH)], src_v)
        pltpu.sync_copy(e_hbm.at[1, w, pl.ds(h * NCHH, NCHH)], dst_v)
        for b in range(NB):
            pltpu.async_copy(g_hbm.at[src_v.at[b]], bufs[b], gsem[b])

        def outer(jo, carry):
            for b in range(NB):
                jj = jo * NB + b
                pltpu.make_async_copy(g_hbm.at[src_v.at[jj]], bufs[b],
                                      gsem[b]).wait()
                pltpu.sync_copy(bufs[b], acc_sh.at[dst_v.at[jj]], add=True)
                pltpu.async_copy(g_hbm.at[src_v.at[jj + NB]], bufs[b], gsem[b])
            return carry

        lax.fori_loop(0, NCHH // NB - 1, outer, 0)
        for b in range(NB):
            jj = NCHH - NB + b
            pltpu.make_async_copy(g_hbm.at[src_v.at[jj]], bufs[b],
                                  gsem[b]).wait()
            pltpu.sync_copy(bufs[b], acc_sh.at[dst_v.at[jj]], add=True)
    plsc.subcore_barrier()
    pltpu.sync_copy(acc_sh.at[pl.ds(s * RPT, RPT)],
                    out_hbm.at[c, pl.ds(s * RPT, RPT)])


def _enc_body(x_ref, we_ref, be_ref, wc0_ref, z0_ref):
    h0 = jnp.maximum(
        jnp.dot(x_ref[...], we_ref[...], preferred_element_type=jnp.float32)
        + be_ref[...], 0.0)
    z0_ref[...] = jnp.dot(h0, wc0_ref[...], preferred_element_type=jnp.float32)


def _scale_body(degp_ref, z0_ref, g0_ref, dis_ref):
    deg = degp_ref[0, :, 0:1] + degp_ref[1, :, 0:1] + 1.0
    row = lax.broadcasted_iota(jnp.int32, (PN, 1), 0)
    dis = jnp.where(row < N, lax.rsqrt(deg), 0.0)
    g0_ref[...] = dis * z0_ref[...]
    dis_ref[...] = dis


def _mid_body(accp_ref, g_ref, dis_ref, b_ref, wn_ref, gn_ref):
    dis = dis_ref[...]
    agg = accp_ref[0] + accp_ref[1] + g_ref[...]
    h = jnp.maximum(dis * agg + b_ref[...], 0.0)
    gn_ref[...] = dis * jnp.dot(h, wn_ref[...],
                                preferred_element_type=jnp.float32)


def _final_body(accp_ref, g_ref, dis_ref, b_ref, wd_ref, bd_ref, out_ref):
    dis = dis_ref[...]
    agg = accp_ref[0] + accp_ref[1] + g_ref[...]
    h = jnp.maximum(dis * agg + b_ref[...], 0.0)
    out_ref[...] = jnp.dot(h, wd_ref[...],
                           preferred_element_type=jnp.float32) + bd_ref[...]


def _tc(body, out_shapes, *args):
    return pl.pallas_call(body, out_shape=out_shapes)(*args)


def kernel(x, edge_index, W_enc, b_enc, W_conv, b_conv, W_dec, b_dec):
    f32 = jnp.float32
    L = W_conv.shape[0]
    d_out = W_dec.shape[1]

    e_t = edge_index.shape[1] // NW
    real = edge_index.reshape(2, NW, e_t)
    padrow = (N + jnp.arange(E_TILE - e_t, dtype=jnp.int32) % (PN - N))
    pads = jnp.broadcast_to(padrow, (2, NW, E_TILE - e_t))
    e3 = jnp.concatenate([real, pads], axis=2).reshape(2, NW, NCH, K)
    xp = jnp.pad(x, ((0, PN - N), (0, 0)))
    zeros = jnp.zeros((PN, H), f32)
    row = jnp.arange(PN, dtype=jnp.int32)[:, None]
    ones_g = jnp.where(row < N, 1.0, 0.0) * jnp.ones((PN, H), f32)
    be2 = b_enc.reshape(1, H)
    wd_pad = jnp.pad(W_dec, ((0, 0), (0, H - d_out)))
    bd_pad = jnp.pad(b_dec, (0, H - d_out)).reshape(1, H)

    degp = _sc_spmm(ones_g, e3, zeros)
    z0 = _tc(_enc_body, jax.ShapeDtypeStruct((PN, H), f32),
             xp, W_enc, be2, W_conv[0])
    g, dis = _tc(
        _scale_body,
        (jax.ShapeDtypeStruct((PN, H), f32), jax.ShapeDtypeStruct((PN, 1), f32)),
        degp, z0)

    for i in range(L):
        accp = _sc_spmm(g, e3, zeros)
        bi = b_conv[i].reshape(1, H)
        if i + 1 < L:
            g = _tc(_mid_body, jax.ShapeDtypeStruct((PN, H), f32),
                    accp, g, dis, bi, W_conv[i + 1])
        else:
            outp = _tc(_final_body, jax.ShapeDtypeStruct((PN, H), f32),
                       accp, g, dis, bi, wd_pad, bd_pad)
    return outp[:N, :d_out]


# speedup vs baseline: 1.3655x; 1.0783x over previous
import functools

import jax
import jax.numpy as jnp
from jax import lax
from jax.experimental import pallas as pl
from jax.experimental.pallas import tpu as pltpu
from jax.experimental.pallas import tpu_sc as plsc

N = 10000
H = 128
NC = 2
NS = 16
NW = NC * NS
K = 128
NCH = 80
NCHH = NCH // 2
E_TILE = NCH * K
E_PAD = NW * E_TILE
PN = 10112
RPT = PN // NS

_mesh = plsc.VectorSubcoreMesh(core_axis_name="c", subcore_axis_name="s")


NB = 2


@functools.partial(
    pl.kernel,
    mesh=_mesh,
    out_type=jax.ShapeDtypeStruct((NC, PN, H), jnp.float32),
    scratch_types=[
        pltpu.VMEM_SHARED((PN, H), jnp.float32),
        pltpu.VMEM((NCHH, K), jnp.int32),
        pltpu.VMEM((NCHH, K), jnp.int32),
        pltpu.VMEM((K, H), jnp.float32),
        pltpu.VMEM((K, H), jnp.float32),
        pltpu.SemaphoreType.DMA,
        pltpu.SemaphoreType.DMA,
    ],
)
def _sc_spmm(g_hbm, e_hbm, z_hbm, out_hbm, acc_sh, src_v, dst_v, b0, b1,
             g0, g1):
    bufs = [b0, b1]
    gsem = [g0, g1]
    c = lax.axis_index("c")
    s = lax.axis_index("s")
    w = c * NS + s
    pltpu.sync_copy(z_hbm.at[pl.ds(s * RPT, RPT)], acc_sh.at[pl.ds(s * RPT, RPT)])
    plsc.subcore_barrier()

    for h in range(2):
        pltpu.sync_copy(e_hbm.at[0, w, pl.ds(h * NCHH, NCHH)], src_v)
        pltpu.sync_copy(e_hbm.at[1, w, pl.ds(h * NCHH, NCHH)], dst_v)
        for b in range(NB):
            pltpu.async_copy(g_hbm.at[src_v.at[b]], bufs[b], gsem[b])

        def outer(jo, carry):
            for b in range(NB):
                jj = jo * NB + b
                pltpu.make_async_copy(g_hbm.at[src_v.at[jj]], bufs[b],
                                      gsem[b]).wait()
                pltpu.sync_copy(bufs[b], acc_sh.at[dst_v.at[jj]], add=True)
                pltpu.async_copy(g_hbm.at[src_v.at[jj + NB]], bufs[b], gsem[b])
            return carry

        lax.fori_loop(0, NCHH // NB - 1, outer, 0)
        for b in range(NB):
            jj = NCHH - NB + b
            pltpu.make_async_copy(g_hbm.at[src_v.at[jj]], bufs[b],
                                  gsem[b]).wait()
            pltpu.sync_copy(bufs[b], acc_sh.at[dst_v.at[jj]], add=True)
    plsc.subcore_barrier()
    pltpu.sync_copy(acc_sh.at[pl.ds(s * RPT, RPT)],
                    out_hbm.at[c, pl.ds(s * RPT, RPT)])


@functools.partial(
    pl.kernel,
    mesh=_mesh,
    out_type=jax.ShapeDtypeStruct((NC, PN, H), jnp.float32),
    scratch_types=[
        pltpu.VMEM_SHARED((PN, H), jnp.float32),
        pltpu.VMEM((NCH, K), jnp.int32),
        pltpu.VMEM((K, H), jnp.float32),
        pltpu.SemaphoreType.DMA,
        pltpu.SemaphoreType.DMA,
    ],
)
def _sc_degree(e_hbm, ones_hbm, z_hbm, out_hbm, acc_sh, dst_v, ones_v,
               s0, s1):
    sems = [s0, s1]
    c = lax.axis_index("c")
    s = lax.axis_index("s")
    w = c * NS + s
    pltpu.sync_copy(z_hbm.at[pl.ds(s * RPT, RPT)], acc_sh.at[pl.ds(s * RPT, RPT)])
    pltpu.sync_copy(e_hbm.at[1, w], dst_v)
    pltpu.sync_copy(ones_hbm, ones_v)
    plsc.subcore_barrier()

    for b in range(2):
        pltpu.async_copy(ones_v, acc_sh.at[dst_v.at[b]], sems[b], add=True)

    def outer(jo, carry):
        for b in range(2):
            jj = jo * 2 + b
            pltpu.make_async_copy(ones_v, acc_sh.at[dst_v.at[jj]],
                                  sems[b]).wait()
            pltpu.async_copy(ones_v, acc_sh.at[dst_v.at[jj + 2]], sems[b],
                             add=True)
        return carry

    lax.fori_loop(0, NCH // 2 - 1, outer, 0)
    for b in range(2):
        jj = NCH - 2 + b
        pltpu.make_async_copy(ones_v, acc_sh.at[dst_v.at[jj]], sems[b]).wait()
    plsc.subcore_barrier()
    pltpu.sync_copy(acc_sh.at[pl.ds(s * RPT, RPT)],
                    out_hbm.at[c, pl.ds(s * RPT, RPT)])


def _prep_body(x_ref, we_ref, be_ref, wc0_ref, degp_ref, g0_ref, dis_ref):
    deg = degp_ref[0, :, 0:1] + degp_ref[1, :, 0:1] + 1.0
    row = lax.broadcasted_iota(jnp.int32, (PN, 1), 0)
    dis = jnp.where(row < N, lax.rsqrt(deg), 0.0)
    h0 = jnp.maximum(
        jnp.dot(x_ref[...], we_ref[...], preferred_element_type=jnp.float32)
        + be_ref[...], 0.0)
    g0_ref[...] = dis * jnp.dot(h0, wc0_ref[...],
                                preferred_element_type=jnp.float32)
    dis_ref[...] = dis


def _mid_body(accp_ref, g_ref, dis_ref, b_ref, wn_ref, gn_ref):
    dis = dis_ref[...]
    agg = accp_ref[0] + accp_ref[1] + g_ref[...]
    h = jnp.maximum(dis * agg + b_ref[...], 0.0)
    gn_ref[...] = dis * jnp.dot(h, wn_ref[...],
                                preferred_element_type=jnp.float32)


def _final_body(accp_ref, g_ref, dis_ref, b_ref, wd_ref, bd_ref, out_ref):
    dis = dis_ref[...]
    agg = accp_ref[0] + accp_ref[1] + g_ref[...]
    h = jnp.maximum(dis * agg + b_ref[...], 0.0)
    out_ref[...] = jnp.dot(h, wd_ref[...],
                           preferred_element_type=jnp.float32) + bd_ref[...]


def _tc(body, out_shapes, *args):
    return pl.pallas_call(body, out_shape=out_shapes)(*args)


def kernel(x, edge_index, W_enc, b_enc, W_conv, b_conv, W_dec, b_dec):
    f32 = jnp.float32
    L = W_conv.shape[0]
    d_out = W_dec.shape[1]

    e_t = edge_index.shape[1] // NW
    real = edge_index.reshape(2, NW, e_t)
    padrow = (N + jnp.arange(E_TILE - e_t, dtype=jnp.int32) % (PN - N))
    pads = jnp.broadcast_to(padrow, (2, NW, E_TILE - e_t))
    e3 = jnp.concatenate([real, pads], axis=2).reshape(2, NW, NCH, K)
    xp = jnp.pad(x, ((0, PN - N), (0, 0)))
    zeros = jnp.zeros((PN, H), f32)
    ones_k = jnp.ones((K, H), f32)
    be2 = b_enc.reshape(1, H)
    wd_pad = jnp.pad(W_dec, ((0, 0), (0, H - d_out)))
    bd_pad = jnp.pad(b_dec, (0, H - d_out)).reshape(1, H)

    degp = _sc_degree(e3, ones_k, zeros)
    g, dis = _tc(
        _prep_body,
        (jax.ShapeDtypeStruct((PN, H), f32), jax.ShapeDtypeStruct((PN, 1), f32)),
        xp, W_enc, be2, W_conv[0], degp)

    for i in range(L):
        accp = _sc_spmm(g, e3, zeros)
        bi = b_conv[i].reshape(1, H)
        if i + 1 < L:
            g = _tc(_mid_body, jax.ShapeDtypeStruct((PN, H), f32),
                    accp, g, dis, bi, W_conv[i + 1])
        else:
            outp = _tc(_final_body, jax.ShapeDtypeStruct((PN, H), f32),
                       accp, g, dis, bi, wd_pad, bd_pad)
    return outp[:N, :d_out]


# speedup vs baseline: 1.4722x; 1.0781x over previous
import functools

import jax
import jax.numpy as jnp
from jax import lax
from jax.experimental import pallas as pl
from jax.experimental.pallas import tpu as pltpu
from jax.experimental.pallas import tpu_sc as plsc

N = 10000
H = 128
NC = 2
NS = 16
NW = NC * NS
K = 128
NCH = 80
NCHH = NCH // 2
E_TILE = NCH * K
E_PAD = NW * E_TILE
PN = 10112
RPT = PN // NS

_mesh = plsc.VectorSubcoreMesh(core_axis_name="c", subcore_axis_name="s")


NB = 2


@functools.partial(
    pl.kernel,
    mesh=_mesh,
    out_type=jax.ShapeDtypeStruct((NC, PN, H), jnp.float32),
    scratch_types=[
        pltpu.VMEM_SHARED((PN, H), jnp.float32),
        pltpu.VMEM((NCHH, K), jnp.int32),
        pltpu.VMEM((NCHH, K), jnp.int32),
        pltpu.VMEM((K, H), jnp.float32),
        pltpu.VMEM((K, H), jnp.float32),
        pltpu.SemaphoreType.DMA,
        pltpu.SemaphoreType.DMA,
    ],
)
def _sc_spmm(g_hbm, e_hbm, z_hbm, out_hbm, acc_sh, src_v, dst_v, b0, b1,
             g0, g1):
    bufs = [b0, b1]
    gsem = [g0, g1]
    c = lax.axis_index("c")
    s = lax.axis_index("s")
    w = c * NS + s
    pltpu.sync_copy(z_hbm.at[pl.ds(s * RPT, RPT)], acc_sh.at[pl.ds(s * RPT, RPT)])
    plsc.subcore_barrier()

    for h in range(2):
        pltpu.sync_copy(e_hbm.at[0, w, pl.ds(h * NCHH, NCHH)], src_v)
        pltpu.sync_copy(e_hbm.at[1, w, pl.ds(h * NCHH, NCHH)], dst_v)
        for b in range(NB):
            pltpu.async_copy(g_hbm.at[src_v.at[b]], bufs[b], gsem[b])

        def outer(jo, carry):
            for b in range(NB):
                jj = jo * NB + b
                pltpu.make_async_copy(g_hbm.at[src_v.at[jj]], bufs[b],
                                      gsem[b]).wait()
                pltpu.sync_copy(bufs[b], acc_sh.at[dst_v.at[jj]], add=True)
                pltpu.async_copy(g_hbm.at[src_v.at[jj + NB]], bufs[b], gsem[b])
            return carry

        lax.fori_loop(0, NCHH // NB - 1, outer, 0)
        for b in range(NB):
            jj = NCHH - NB + b
            pltpu.make_async_copy(g_hbm.at[src_v.at[jj]], bufs[b],
                                  gsem[b]).wait()
            pltpu.sync_copy(bufs[b], acc_sh.at[dst_v.at[jj]], add=True)
    plsc.subcore_barrier()
    pltpu.sync_copy(acc_sh.at[pl.ds(s * RPT, RPT)],
                    out_hbm.at[c, pl.ds(s * RPT, RPT)])


NBLK = PN // K


@functools.partial(
    pl.kernel,
    mesh=_mesh,
    out_type=jax.ShapeDtypeStruct((NC * PN,), jnp.float32),
    scratch_types=[
        pltpu.VMEM_SHARED((PN,), jnp.float32),
        pltpu.VMEM((NCH, K), jnp.int32),
        pltpu.VMEM((K,), jnp.float32),
        pltpu.VMEM((K,), jnp.float32),
        pltpu.SemaphoreType.DMA,
    ],
)
def _sc_degree(e_hbm, ones_hbm, z_hbm, out_hbm, acc_sh, dst_v, ones_v,
               blk_v, sem):
    c = lax.axis_index("c")
    s = lax.axis_index("s")
    w = c * NS + s
    pltpu.sync_copy(z_hbm, blk_v)
    pltpu.sync_copy(e_hbm.at[1, w], dst_v)
    pltpu.sync_copy(ones_hbm, ones_v)
    for i in range(5):
        blk = s + i * NS
        @pl.when(blk < NBLK)
        def _():
            pltpu.sync_copy(blk_v, acc_sh.at[pl.ds(blk * K, K)])
    plsc.subcore_barrier()

    def body(j, carry):
        pltpu.sync_copy(ones_v, acc_sh.at[dst_v.at[j]], add=True)
        return carry

    lax.fori_loop(0, NCH, body, 0)
    plsc.subcore_barrier()
    for i in range(5):
        blk = s + i * NS
        @pl.when(blk < NBLK)
        def _():
            pltpu.sync_copy(acc_sh.at[pl.ds(blk * K, K)], blk_v)
            pltpu.sync_copy(blk_v, out_hbm.at[pl.ds(c * PN + blk * K, K)])


def _prep_body(x_ref, we_ref, be_ref, wc0_ref, degp_ref, g0_ref, dis_ref):
    deg = degp_ref[0] + degp_ref[1] + 1.0
    row = lax.broadcasted_iota(jnp.int32, (PN, 1), 0)
    dis = jnp.where(row < N, lax.rsqrt(deg), 0.0)
    h0 = jnp.maximum(
        jnp.dot(x_ref[...], we_ref[...], preferred_element_type=jnp.float32)
        + be_ref[...], 0.0)
    g0_ref[...] = dis * jnp.dot(h0, wc0_ref[...],
                                preferred_element_type=jnp.float32)
    dis_ref[...] = dis


def _mid_body(accp_ref, g_ref, dis_ref, b_ref, wn_ref, gn_ref):
    dis = dis_ref[...]
    agg = accp_ref[0] + accp_ref[1] + g_ref[...]
    h = jnp.maximum(dis * agg + b_ref[...], 0.0)
    gn_ref[...] = dis * jnp.dot(h, wn_ref[...],
                                preferred_element_type=jnp.float32)


def _final_body(accp_ref, g_ref, dis_ref, b_ref, wd_ref, bd_ref, out_ref):
    dis = dis_ref[...]
    agg = accp_ref[0] + accp_ref[1] + g_ref[...]
    h = jnp.maximum(dis * agg + b_ref[...], 0.0)
    out_ref[...] = jnp.dot(h, wd_ref[...],
                           preferred_element_type=jnp.float32) + bd_ref[...]


def _tc(body, out_shapes, *args):
    return pl.pallas_call(body, out_shape=out_shapes)(*args)


def kernel(x, edge_index, W_enc, b_enc, W_conv, b_conv, W_dec, b_dec):
    f32 = jnp.float32
    L = W_conv.shape[0]
    d_out = W_dec.shape[1]

    e_t = edge_index.shape[1] // NW
    real = edge_index.reshape(2, NW, e_t)
    padrow = (N + jnp.arange(E_TILE - e_t, dtype=jnp.int32) % (PN - N))
    pads = jnp.broadcast_to(padrow, (2, NW, E_TILE - e_t))
    e3 = jnp.concatenate([real, pads], axis=2).reshape(2, NW, NCH, K)
    xp = jnp.pad(x, ((0, PN - N), (0, 0)))
    zeros = jnp.zeros((PN, H), f32)
    ones1 = jnp.ones((K,), f32)
    zeros1 = jnp.zeros((K,), f32)
    be2 = b_enc.reshape(1, H)
    wd_pad = jnp.pad(W_dec, ((0, 0), (0, H - d_out)))
    bd_pad = jnp.pad(b_dec, (0, H - d_out)).reshape(1, H)

    degp = _sc_degree(e3, ones1, zeros1).reshape(NC, PN, 1)
    g, dis = _tc(
        _prep_body,
        (jax.ShapeDtypeStruct((PN, H), f32), jax.ShapeDtypeStruct((PN, 1), f32)),
        xp, W_enc, be2, W_conv[0], degp)

    for i in range(L):
        accp = _sc_spmm(g, e3, zeros)
        bi = b_conv[i].reshape(1, H)
        if i + 1 < L:
            g = _tc(_mid_body, jax.ShapeDtypeStruct((PN, H), f32),
                    accp, g, dis, bi, W_conv[i + 1])
        else:
            outp = _tc(_final_body, jax.ShapeDtypeStruct((PN, H), f32),
                       accp, g, dis, bi, wd_pad, bd_pad)
    return outp[:N, :d_out]


# speedup vs baseline: 1.4745x; 1.0016x over previous
import functools

import jax
import jax.numpy as jnp
from jax import lax
from jax.experimental import pallas as pl
from jax.experimental.pallas import tpu as pltpu
from jax.experimental.pallas import tpu_sc as plsc

N = 10000
H = 128
NC = 2
NS = 16
NW = NC * NS
K = 128
NCH = 80
NCHH = NCH // 2
E_TILE = NCH * K
E_PAD = NW * E_TILE
PN = 10112
RPT = PN // NS

_mesh = plsc.VectorSubcoreMesh(core_axis_name="c", subcore_axis_name="s")


NB = 2


@functools.partial(
    pl.kernel,
    mesh=_mesh,
    out_type=jax.ShapeDtypeStruct((NC, PN, H), jnp.float32),
    scratch_types=[
        pltpu.VMEM_SHARED((PN, H), jnp.float32),
        pltpu.VMEM((NCHH, K), jnp.int32),
        pltpu.VMEM((NCHH, K), jnp.int32),
        pltpu.VMEM((K, H), jnp.float32),
        pltpu.VMEM((K, H), jnp.float32),
        pltpu.SemaphoreType.DMA,
        pltpu.SemaphoreType.DMA,
    ],
)
def _sc_spmm(g_hbm, e_hbm, z_hbm, out_hbm, acc_sh, src_v, dst_v, b0, b1,
             g0, g1):
    bufs = [b0, b1]
    gsem = [g0, g1]
    c = lax.axis_index("c")
    s = lax.axis_index("s")
    w = c * NS + s
    pltpu.sync_copy(z_hbm.at[pl.ds(s * RPT, RPT)], acc_sh.at[pl.ds(s * RPT, RPT)])
    plsc.subcore_barrier()

    for h in range(2):
        pltpu.sync_copy(e_hbm.at[0, w, pl.ds(h * NCHH, NCHH)], src_v)
        pltpu.sync_copy(e_hbm.at[1, w, pl.ds(h * NCHH, NCHH)], dst_v)
        for b in range(NB):
            pltpu.async_copy(g_hbm.at[src_v.at[b]], bufs[b], gsem[b])

        def outer(jo, carry):
            for b in range(NB):
                jj = jo * NB + b
                pltpu.make_async_copy(g_hbm.at[src_v.at[jj]], bufs[b],
                                      gsem[b]).wait()
                pltpu.sync_copy(bufs[b], acc_sh.at[dst_v.at[jj]], add=True)
                pltpu.async_copy(g_hbm.at[src_v.at[jj + NB]], bufs[b], gsem[b])
            return carry

        lax.fori_loop(0, NCHH // NB - 1, outer, 0)
        for b in range(NB):
            jj = NCHH - NB + b
            pltpu.make_async_copy(g_hbm.at[src_v.at[jj]], bufs[b],
                                  gsem[b]).wait()
            pltpu.sync_copy(bufs[b], acc_sh.at[dst_v.at[jj]], add=True)
    plsc.subcore_barrier()
    pltpu.sync_copy(acc_sh.at[pl.ds(s * RPT, RPT)],
                    out_hbm.at[c, pl.ds(s * RPT, RPT)])


NBLK = PN // K


@functools.partial(
    pl.kernel,
    mesh=_mesh,
    out_type=jax.ShapeDtypeStruct((NC * PN,), jnp.float32),
    scratch_types=[
        pltpu.VMEM_SHARED((PN,), jnp.float32),
        pltpu.VMEM((NCH, K), jnp.int32),
        pltpu.VMEM((K,), jnp.float32),
        pltpu.VMEM((K,), jnp.float32),
        pltpu.SemaphoreType.DMA,
    ],
)
def _sc_degree(e_hbm, ones_hbm, z_hbm, out_hbm, acc_sh, dst_v, ones_v,
               blk_v, sem):
    c = lax.axis_index("c")
    s = lax.axis_index("s")
    w = c * NS + s
    pltpu.sync_copy(z_hbm, blk_v)
    pltpu.sync_copy(e_hbm.at[1, w], dst_v)
    pltpu.sync_copy(ones_hbm, ones_v)
    for i in range(5):
        blk = s + i * NS
        @pl.when(blk < NBLK)
        def _():
            pltpu.sync_copy(blk_v, acc_sh.at[pl.ds(blk * K, K)])
    plsc.subcore_barrier()

    def body(j, carry):
        pltpu.sync_copy(ones_v, acc_sh.at[dst_v.at[j]], add=True)
        return carry

    lax.fori_loop(0, NCH, body, 0)
    plsc.subcore_barrier()
    for i in range(5):
        blk = s + i * NS
        @pl.when(blk < NBLK)
        def _():
            pltpu.sync_copy(acc_sh.at[pl.ds(blk * K, K)], blk_v)
            pltpu.sync_copy(blk_v, out_hbm.at[pl.ds(c * PN + blk * K, K)])


def _prep_body(x_ref, we_ref, be_ref, wc0_ref, degp_ref, g0_ref, dis_ref):
    blk = pl.program_id(0)
    deg = degp_ref[0] + degp_ref[1] + 1.0
    row = lax.broadcasted_iota(jnp.int32, (BR, 1), 0) + blk * BR
    dis = jnp.where(row < N, lax.rsqrt(deg), 0.0)
    h0 = jnp.maximum(
        jnp.dot(x_ref[...], we_ref[...], preferred_element_type=jnp.float32)
        + be_ref[...], 0.0)
    g0_ref[...] = dis * jnp.dot(h0, wc0_ref[...],
                                preferred_element_type=jnp.float32)
    dis_ref[...] = dis


def _mid_body(accp_ref, g_ref, dis_ref, b_ref, wn_ref, gn_ref):
    dis = dis_ref[...]
    agg = accp_ref[0] + accp_ref[1] + g_ref[...]
    h = jnp.maximum(dis * agg + b_ref[...], 0.0)
    gn_ref[...] = dis * jnp.dot(h, wn_ref[...],
                                preferred_element_type=jnp.float32)


def _final_body(accp_ref, g_ref, dis_ref, b_ref, wd_ref, bd_ref, out_ref):
    dis = dis_ref[...]
    agg = accp_ref[0] + accp_ref[1] + g_ref[...]
    h = jnp.maximum(dis * agg + b_ref[...], 0.0)
    out_ref[...] = (jnp.dot(h, wd_ref[...],
                            preferred_element_type=jnp.float32)
                    + bd_ref[...])[:N]


def _tc(body, out_shapes, *args):
    return pl.pallas_call(body, out_shape=out_shapes)(*args)


GR = 8
BR = PN // GR

_prep_call = pl.pallas_call(
    _prep_body,
    grid=(GR,),
    in_specs=[
        pl.BlockSpec((BR, H), lambda i: (i, 0)),
        pl.BlockSpec((H, H), lambda i: (0, 0)),
        pl.BlockSpec((1, H), lambda i: (0, 0)),
        pl.BlockSpec((H, H), lambda i: (0, 0)),
        pl.BlockSpec((NC, BR, 1), lambda i: (0, i, 0)),
    ],
    out_specs=(pl.BlockSpec((BR, H), lambda i: (i, 0)),
               pl.BlockSpec((BR, 1), lambda i: (i, 0))),
    out_shape=(jax.ShapeDtypeStruct((PN, H), jnp.float32),
               jax.ShapeDtypeStruct((PN, 1), jnp.float32)),
)

_mid_call = pl.pallas_call(
    _mid_body,
    grid=(GR,),
    in_specs=[
        pl.BlockSpec((NC, BR, H), lambda i: (0, i, 0)),
        pl.BlockSpec((BR, H), lambda i: (i, 0)),
        pl.BlockSpec((BR, 1), lambda i: (i, 0)),
        pl.BlockSpec((1, H), lambda i: (0, 0)),
        pl.BlockSpec((H, H), lambda i: (0, 0)),
    ],
    out_specs=pl.BlockSpec((BR, H), lambda i: (i, 0)),
    out_shape=jax.ShapeDtypeStruct((PN, H), jnp.float32),
)


def kernel(x, edge_index, W_enc, b_enc, W_conv, b_conv, W_dec, b_dec):
    f32 = jnp.float32
    L = W_conv.shape[0]
    d_out = W_dec.shape[1]

    e_t = edge_index.shape[1] // NW
    real = edge_index.reshape(2, NW, e_t)
    padrow = (N + jnp.arange(E_TILE - e_t, dtype=jnp.int32) % (PN - N))
    pads = jnp.broadcast_to(padrow, (2, NW, E_TILE - e_t))
    e3 = jnp.concatenate([real, pads], axis=2).reshape(2, NW, NCH, K)
    xp = jnp.pad(x, ((0, PN - N), (0, 0)))
    zeros = jnp.zeros((PN, H), f32)
    ones1 = jnp.ones((K,), f32)
    zeros1 = jnp.zeros((K,), f32)
    be2 = b_enc.reshape(1, H)
    bd2 = b_dec.reshape(1, d_out)

    degp = _sc_degree(e3, ones1, zeros1).reshape(NC, PN, 1)
    g, dis = _prep_call(xp, W_enc, be2, W_conv[0], degp)

    for i in range(L):
        accp = _sc_spmm(g, e3, zeros)
        bi = b_conv[i].reshape(1, H)
        if i + 1 < L:
            g = _mid_call(accp, g, dis, bi, W_conv[i + 1])
        else:
            out = _tc(_final_body, jax.ShapeDtypeStruct((N, d_out), f32),
                      accp, g, dis, bi, W_dec, bd2)
    return out
